# baseline (speedup 1.0000x reference)
import functools

import jax
import jax.numpy as jnp
from jax import lax
from jax.experimental import pallas as pl
from jax.experimental.pallas import tpu as pltpu
from jax.experimental.pallas import tpu_sc as plsc

D = 128
K = 16
V = 64
B = 512
LEVELS = 7
NC = 2
NS = 16
NW = NC * NS
DPW = D // NW
LANES = 16


CB = 128
NCHUNK = B // CB


def _gather_body(x_ref, tab_ref, out_ref, xva, xvb, tabv, outv, sem0, sem1):
    wid = lax.axis_index("s") * NC + lax.axis_index("c")
    d0 = wid * DPW
    pltpu.sync_copy(tab_ref.at[pl.ds(d0 * (K * V), DPW * K * V)], tabv)
    ib_d = lax.iota(jnp.int32, LANES) * D
    sems = (sem0, sem1)
    bufs = (xva, xvb)
    copies = [pltpu.async_copy(x_ref.at[pl.ds(0, CB * D)], xva, sem0)]
    for cb in range(NCHUNK):
        buf = bufs[cb % 2]
        copies[cb].wait()
        if cb + 1 < NCHUNK:
            copies.append(pltpu.async_copy(
                x_ref.at[pl.ds((cb + 1) * CB * D, CB * D)],
                bufs[(cb + 1) % 2], sems[(cb + 1) % 2]))

        @plsc.parallel_loop(0, CB // LANES, unroll=4)
        def _chunk(c, cb=cb, buf=buf):
            for dl in range(DPW):
                xb = plsc.load_gather(buf,
                                      [ib_d + (c * (LANES * D) + d0 + dl)])
                for k in range(K):
                    idx = xb + (dl * K * V + k * V)
                    outv[pl.ds(dl * K * B + k * B + cb * CB + c * LANES,
                               LANES)] = plsc.load_gather(tabv, [idx])

    pltpu.sync_copy(outv, out_ref.at[pl.ds(wid * (DPW * K * B), DPW * K * B)])


def _make_sc_gather(interpret):
    return pl.kernel(
        _gather_body,
        out_type=jax.ShapeDtypeStruct((D * K * B,), jnp.float32),
        mesh=plsc.VectorSubcoreMesh(core_axis_name="c", subcore_axis_name="s",
                                    num_cores=NC, num_subcores=NS),
        scratch_types=[
            pltpu.VMEM((CB * D,), jnp.int32),
            pltpu.VMEM((CB * D,), jnp.int32),
            pltpu.VMEM((DPW * K * V,), jnp.float32),
            pltpu.VMEM((DPW * K * B,), jnp.float32),
            pltpu.SemaphoreType.DMA,
            pltpu.SemaphoreType.DMA,
        ],
        compiler_params=pltpu.CompilerParams(needs_layout_passes=False),
        interpret=interpret,
    )


def _levels_kernel(node_ref, logits_ref, w0, w1, w2, w3, w4, w5, w6, rw_ref,
                   out_ref):
    ws = (w0, w1, w2, w3, w4, w5, w6)
    wn = []
    for w_ref in ws:
        w = w_ref[...]
        m = jnp.max(w, axis=-1, keepdims=True)
        e = jnp.exp(w - m)
        wn.append(e / jnp.sum(e, axis=-1, keepdims=True))
    logits = logits_ref[...]
    lmax = jnp.max(logits, axis=-1, keepdims=True)
    logz = jnp.log(jnp.sum(jnp.exp(logits - lmax), axis=-1, keepdims=True)) + lmax
    rw = rw_ref[...]
    rm = jnp.max(rw, axis=-1, keepdims=True)
    re = jnp.exp(rw - rm)
    rwn = re / jnp.sum(re, axis=-1, keepdims=True)

    mars = node_ref[...] - logz
    for l in range(LEVELS):
        r = mars.shape[0] // 2
        m4 = mars.reshape(r, 2, K, B)
        left = m4[:, 0]
        right = m4[:, 1]
        mxl = jnp.max(left, axis=1, keepdims=True)
        mxr = jnp.max(right, axis=1, keepdims=True)
        el = jnp.exp(left - mxl)
        er = jnp.exp(right - mxr)
        p = (el[:, :, None, :] * er[:, None, :, :]).reshape(r, K * K, B)
        lin = lax.dot_general(
            wn[l], p, (((2,), (1,)), ((0,), (0,))),
            preferred_element_type=jnp.float32)
        mars = jnp.log(lin) + mxl + mxr
    m0 = mars[0]
    mx = jnp.max(m0, axis=0, keepdims=True)
    e0 = jnp.exp(m0 - mx)
    out_ref[...] = jnp.log(
        jnp.dot(rwn, e0, preferred_element_type=jnp.float32)) + mx


@functools.partial(jax.jit, static_argnames=("interpret",))
def kernel(inputs, input_logits, w0, w1, w2, w3, w4, w5, w6, root_w,
           interpret=False):
    x_flat = inputs.reshape(-1)
    logits_flat = input_logits.reshape(-1)
    node_flat = _make_sc_gather(interpret)(x_flat, logits_flat)
    node = node_flat.reshape(D, K, B)
    out = pl.pallas_call(
        _levels_kernel,
        out_shape=jax.ShapeDtypeStruct((1, B), jnp.float32),
        interpret=interpret,
    )(node, input_logits, w0, w1, w2, w3, w4, w5, w6, root_w.reshape(1, K))
    return out.reshape(B)


# speedup vs baseline: 1.2352x; 1.2352x over previous
import functools

import jax
import jax.numpy as jnp
from jax import lax
from jax.experimental import pallas as pl
from jax.experimental.pallas import tpu as pltpu
from jax.experimental.pallas import tpu_sc as plsc

D = 128
K = 16
V = 64
B = 512
LEVELS = 7
NC = 2
NS = 16
NW = NC * NS
DPW = D // NW
LANES = 16


def _gather_body(x_ref, tab_ref, out_ref, xv, tabv, outv, sem):
    wid = lax.axis_index("s") * NC + lax.axis_index("c")
    pltpu.sync_copy(x_ref.at[pl.ds(wid * (DPW * B), DPW * B)], xv)
    pltpu.sync_copy(tab_ref.at[pl.ds(wid * (DPW * K * V), DPW * K * V)], tabv)

    copies = []
    for dl in range(DPW):
        @plsc.parallel_loop(0, B // LANES, unroll=4)
        def _chunk(c, dl=dl):
            xb = xv[pl.ds(dl * B + c * LANES, LANES)]
            for k in range(K):
                idx = xb + (dl * K * V + k * V)
                outv[pl.ds(dl * K * B + k * B + c * LANES, LANES)] = (
                    plsc.load_gather(tabv, [idx]))

        copies.append(pltpu.async_copy(
            outv.at[pl.ds(dl * K * B, K * B)],
            out_ref.at[pl.ds(wid * (DPW * K * B) + dl * K * B, K * B)],
            sem))
    for c in copies:
        c.wait()


def _make_sc_gather(interpret):
    return pl.kernel(
        _gather_body,
        out_type=jax.ShapeDtypeStruct((D * K * B,), jnp.float32),
        mesh=plsc.VectorSubcoreMesh(core_axis_name="c", subcore_axis_name="s",
                                    num_cores=NC, num_subcores=NS),
        scratch_types=[
            pltpu.VMEM((DPW * B,), jnp.int32),
            pltpu.VMEM((DPW * K * V,), jnp.float32),
            pltpu.VMEM((DPW * K * B,), jnp.float32),
            pltpu.SemaphoreType.DMA,
        ],
        compiler_params=pltpu.CompilerParams(needs_layout_passes=False),
        interpret=interpret,
    )


def _levels_kernel(node_ref, logits_ref, w0, w1, w2, w3, w4, w5, w6, rw_ref,
                   out_ref):
    ws = (w0, w1, w2, w3, w4, w5, w6)
    wn = []
    for w_ref in ws:
        w = w_ref[...]
        m = jnp.max(w, axis=-1, keepdims=True)
        e = jnp.exp(w - m)
        wn.append(e / jnp.sum(e, axis=-1, keepdims=True))
    logits = logits_ref[...]
    lmax = jnp.max(logits, axis=-1, keepdims=True)
    logz = jnp.log(jnp.sum(jnp.exp(logits - lmax), axis=-1, keepdims=True)) + lmax
    rw = rw_ref[...]
    rm = jnp.max(rw, axis=-1, keepdims=True)
    re = jnp.exp(rw - rm)
    rwn = re / jnp.sum(re, axis=-1, keepdims=True)

    mars = node_ref[...] - logz
    for l in range(LEVELS):
        r = mars.shape[0] // 2
        m4 = mars.reshape(r, 2, K, B)
        left = m4[:, 0]
        right = m4[:, 1]
        mxl = jnp.max(left, axis=1, keepdims=True)
        mxr = jnp.max(right, axis=1, keepdims=True)
        el = jnp.exp(left - mxl)
        er = jnp.exp(right - mxr)
        p = (el[:, :, None, :] * er[:, None, :, :]).reshape(r, K * K, B)
        lin = lax.dot_general(
            wn[l], p, (((2,), (1,)), ((0,), (0,))),
            preferred_element_type=jnp.float32)
        mars = jnp.log(lin) + mxl + mxr
    m0 = mars[0]
    mx = jnp.max(m0, axis=0, keepdims=True)
    e0 = jnp.exp(m0 - mx)
    out_ref[...] = jnp.log(
        jnp.dot(rwn, e0, preferred_element_type=jnp.float32)) + mx


@functools.partial(jax.jit, static_argnames=("interpret",))
def kernel(inputs, input_logits, w0, w1, w2, w3, w4, w5, w6, root_w,
           interpret=False):
    x_flat = inputs.T.reshape(-1)
    logits_flat = input_logits.reshape(-1)
    node_flat = _make_sc_gather(interpret)(x_flat, logits_flat)
    node = node_flat.reshape(D, K, B)
    out = pl.pallas_call(
        _levels_kernel,
        out_shape=jax.ShapeDtypeStruct((1, B), jnp.float32),
        interpret=interpret,
    )(node, input_logits, w0, w1, w2, w3, w4, w5, w6, root_w.reshape(1, K))
    return out.reshape(B)


# speedup vs baseline: 1.2361x; 1.0008x over previous
import functools

import jax
import jax.numpy as jnp
from jax import lax
from jax.experimental import pallas as pl
from jax.experimental.pallas import tpu as pltpu
from jax.experimental.pallas import tpu_sc as plsc

D = 128
K = 16
V = 64
B = 512
LEVELS = 7
NC = 2
NS = 16
NW = NC * NS
DPW = D // NW
LANES = 16


def _gather_body(x_ref, tab_ref, out_ref, xv, tabv, outv, sem):
    wid = lax.axis_index("s") * NC + lax.axis_index("c")
    pltpu.sync_copy(x_ref.at[pl.ds(wid * (DPW * B), DPW * B)], xv)
    pltpu.sync_copy(tab_ref.at[pl.ds(wid * (DPW * K * V), DPW * K * V)], tabv)

    copies = []
    for dl in range(DPW):
        @plsc.parallel_loop(0, B // LANES, unroll=4)
        def _chunk(c, dl=dl):
            xb = xv[pl.ds(dl * B + c * LANES, LANES)]
            for k in range(K):
                idx = xb + (dl * K * V + k * V)
                outv[pl.ds(dl * K * B + k * B + c * LANES, LANES)] = (
                    plsc.load_gather(tabv, [idx]))

        copies.append(pltpu.async_copy(
            outv.at[pl.ds(dl * K * B, K * B)],
            out_ref.at[pl.ds(wid * (DPW * K * B) + dl * K * B, K * B)],
            sem))
    for c in copies:
        c.wait()


def _make_sc_gather(interpret):
    return pl.kernel(
        _gather_body,
        out_type=jax.ShapeDtypeStruct((D * K * B,), jnp.float32),
        mesh=plsc.VectorSubcoreMesh(core_axis_name="c", subcore_axis_name="s",
                                    num_cores=NC, num_subcores=NS),
        scratch_types=[
            pltpu.VMEM((DPW * B,), jnp.int32),
            pltpu.VMEM((DPW * K * V,), jnp.float32),
            pltpu.VMEM((DPW * K * B,), jnp.float32),
            pltpu.SemaphoreType.DMA,
        ],
        compiler_params=pltpu.CompilerParams(needs_layout_passes=False),
        interpret=interpret,
    )


RTOT = D - 1


def _prep_kernel(logits_ref, w0, w1, w2, w3, w4, w5, w6, rw_ref,
                 logz_ref, wn_ref, rwn_ref):
    logits = logits_ref[...]
    lmax = jnp.max(logits, axis=-1, keepdims=True)
    logz = jnp.log(jnp.sum(jnp.exp(logits - lmax), axis=-1, keepdims=True)) + lmax
    logz_ref[...] = jnp.squeeze(logz, axis=-1)
    off = 0
    for w_ref in (w0, w1, w2, w3, w4, w5, w6):
        w = w_ref[...]
        r = w.shape[0]
        m = jnp.max(w, axis=-1, keepdims=True)
        e = jnp.exp(w - m)
        wn_ref[off:off + r] = e / jnp.sum(e, axis=-1, keepdims=True)
        off += r
    rw = rw_ref[...]
    rm = jnp.max(rw, axis=-1, keepdims=True)
    re = jnp.exp(rw - rm)
    rwn_ref[...] = re / jnp.sum(re, axis=-1, keepdims=True)


def _levels_kernel(node_ref, logz_ref, wn_ref, rwn_ref, out_ref):
    mars = node_ref[...] - logz_ref[...][:, :, None]
    off = 0
    for l in range(LEVELS):
        r = mars.shape[0] // 2
        m4 = mars.reshape(r, 2, K, B)
        left = m4[:, 0]
        right = m4[:, 1]
        mxl = jnp.max(left, axis=1, keepdims=True)
        mxr = jnp.max(right, axis=1, keepdims=True)
        el = jnp.exp(left - mxl)
        er = jnp.exp(right - mxr)
        p = (el[:, :, None, :] * er[:, None, :, :]).reshape(r, K * K, B)
        lin = lax.dot_general(
            wn_ref[off:off + r], p, (((2,), (1,)), ((0,), (0,))),
            preferred_element_type=jnp.float32)
        mars = jnp.log(lin) + mxl + mxr
        off += r
    m0 = mars[0]
    mx = jnp.max(m0, axis=0, keepdims=True)
    e0 = jnp.exp(m0 - mx)
    out_ref[...] = jnp.log(
        jnp.dot(rwn_ref[...], e0, preferred_element_type=jnp.float32)) + mx


@functools.partial(jax.jit, static_argnames=("interpret",))
def kernel(inputs, input_logits, w0, w1, w2, w3, w4, w5, w6, root_w,
           interpret=False):
    x_flat = inputs.T.reshape(-1)
    logits_flat = input_logits.reshape(-1)
    node_flat = _make_sc_gather(interpret)(x_flat, logits_flat)
    node = node_flat.reshape(D, K, B)
    logz, wn, rwn = pl.pallas_call(
        _prep_kernel,
        out_shape=(
            jax.ShapeDtypeStruct((D, K), jnp.float32),
            jax.ShapeDtypeStruct((RTOT, K, K * K), jnp.float32),
            jax.ShapeDtypeStruct((1, K), jnp.float32),
        ),
        interpret=interpret,
    )(input_logits, w0, w1, w2, w3, w4, w5, w6, root_w.reshape(1, K))
    out = pl.pallas_call(
        _levels_kernel,
        out_shape=jax.ShapeDtypeStruct((1, B), jnp.float32),
        interpret=interpret,
    )(node, logz, wn, rwn)
    return out.reshape(B)
